# baseline (speedup 1.0000x reference)
import jax
import jax.numpy as jnp
from jax import lax
from jax.experimental import pallas as pl
from jax.experimental.pallas import tpu as pltpu
from jax.experimental.pallas import tpu_sc as plsc

N, E, D, G = 50000, 800000, 128, 256

NPAD = 50176
NB = 8
C = 6272
R = 6400
CAP = 5120
GB = 128
GBLK = CAP // GB
NTILES = 32
EPT = 25600
EROWS = EPT // 128
SB = 40
NROWBLK = NPAD // 128
PR = 384

_MESH = dict(core_axis_name="c", subcore_axis_name="s",
             num_cores=2, num_subcores=16)
_SC_PARAMS = pltpu.CompilerParams(needs_layout_passes=False)


def _sc_deg_body(dst2_h, deg_h, st_d, degv):
    c = lax.axis_index("c")
    s = lax.axis_index("s")
    wid = c * 16 + s
    zeros16 = jnp.zeros((16,), jnp.float32)
    ones16 = jnp.ones((16,), jnp.float32)

    @pl.loop(0, NPAD // 16)
    def _zero(i):
        degv[pl.ds(i * 16, 16)] = zeros16

    for ob in range(EROWS // SB):
        pltpu.sync_copy(dst2_h.at[wid, pl.ds(ob * SB, SB)], st_d)

        @pl.loop(0, SB)
        def _row(r):
            for lane in range(8):
                dv = st_d[r, pl.ds(lane * 16, 16)]
                plsc.addupdate_scatter(degv, [dv], ones16)

    pltpu.sync_copy(degv, deg_h.at[wid])


def _sc_deg(dst2):
    return pl.kernel(
        _sc_deg_body,
        out_type=jax.ShapeDtypeStruct((NTILES, NPAD), jnp.float32),
        mesh=plsc.VectorSubcoreMesh(**_MESH),
        scratch_types=(
            pltpu.VMEM((SB, 128), jnp.int32),
            pltpu.VMEM((NPAD,), jnp.float32),
        ),
        compiler_params=_SC_PARAMS,
    )(dst2)


def _sc_buckets_body(src2, dst2, ls_h, ld_h, cnt_h, st_s, st_d, ls, ld, cbuf):
    c = lax.axis_index("c")
    s = lax.axis_index("s")
    wid = c * 16 + s

    dummy_src = jnp.zeros((16,), jnp.int32)
    dummy_dst = jnp.full((16,), C, jnp.int32)

    offs = tuple(jnp.int32(b * CAP) for b in range(NB))

    for ob in range(EROWS // SB):
        pltpu.sync_copy(src2.at[wid, pl.ds(ob * SB, SB)], st_s)
        pltpu.sync_copy(dst2.at[wid, pl.ds(ob * SB, SB)], st_d)

        @pl.loop(0, SB, init_carry=offs)
        def _row(r, carry):
            offs_in = list(carry)
            for lane in range(8):
                sv = st_s[r, pl.ds(lane * 16, 16)]
                dv = st_d[r, pl.ds(lane * 16, 16)]
                for b in range(NB):
                    lo = b * C
                    m = (dv >= lo) & (dv < lo + C)
                    off = offs_in[b]
                    plsc.store_compressed(ls.at[pl.ds(off, 16)], sv, mask=m)
                    plsc.store_compressed(
                        ld.at[pl.ds(off, 16)], dv - lo, mask=m)
                    cnt = jnp.sum(m.astype(jnp.int32))
                    offs_in[b] = jnp.minimum(off + cnt,
                                             jnp.int32(lo + CAP - 16))
            return tuple(offs_in)

        offs = _row

    full = jnp.ones((16,), jnp.int32) >= 0
    iota16 = lax.iota(jnp.int32, 16)
    cv = jnp.zeros((16,), jnp.int32)
    for b in range(NB):
        off = offs[b]
        for k in range(2 * GB // 16):
            o = jnp.minimum(off + k * 16, jnp.int32(b * CAP + CAP - 16))
            plsc.store_compressed(ls.at[pl.ds(o, 16)], dummy_src, mask=full)
            plsc.store_compressed(ld.at[pl.ds(o, 16)], dummy_dst, mask=full)
        cv = jnp.where(iota16 == b, jnp.full((16,), off - b * CAP), cv)
    cbuf[...] = cv

    pltpu.sync_copy(ls, ls_h.at[wid])
    pltpu.sync_copy(ld, ld_h.at[wid])
    pltpu.sync_copy(cbuf, cnt_h.at[wid])


def _sc_buckets(src2, dst2):
    return pl.kernel(
        _sc_buckets_body,
        out_type=(
            jax.ShapeDtypeStruct((NTILES, NB * CAP), jnp.int32),
            jax.ShapeDtypeStruct((NTILES, NB * CAP), jnp.int32),
            jax.ShapeDtypeStruct((NTILES, 16), jnp.int32),
        ),
        mesh=plsc.VectorSubcoreMesh(**_MESH),
        scratch_types=(
            pltpu.VMEM((SB, 128), jnp.int32),
            pltpu.VMEM((SB, 128), jnp.int32),
            pltpu.VMEM((NB * CAP,), jnp.int32),
            pltpu.VMEM((NB * CAP,), jnp.int32),
            pltpu.VMEM((16,), jnp.int32),
        ),
        compiler_params=_SC_PARAMS,
    )(src2, dst2)


def _sc_agg_body(ht_h, ls4, ld4, cnt_h, agg_h,
                 acc, sidx, didx, r0, r1, zbuf, cbuf, sem0, sem1):
    c = lax.axis_index("c")
    s = lax.axis_index("s")

    zeros16 = jnp.zeros((16,), jnp.float32)
    iota16 = lax.iota(jnp.int32, 16)

    @pl.loop(0, 16 * D // 16)
    def _zb(i):
        zbuf[i // (D // 16), pl.ds((i % (D // 16)) * 16, 16)] = zeros16

    for u in range(NB // 2):
        b = (NB // 2) * c + u

        @pl.loop(0, (R // 16) // 16)
        def _z(i):
            pltpu.sync_copy(zbuf, acc.at[pl.ds(s * (R // 16) + i * 16, 16)])

        plsc.subcore_barrier()
        for u2 in range(2):
            t = 2 * s + u2
            pltpu.sync_copy(cnt_h.at[t], cbuf)
            cntv = cbuf[...]
            cnt = jnp.sum(jnp.where(iota16 == b, cntv, 0))
            npair = (cnt + (2 * GB - 1)) // (2 * GB)

            pltpu.sync_copy(ls4.at[t, b], sidx)
            pltpu.sync_copy(ld4.at[t, b], didx)

            @pl.when(npair > 0)
            def _go():
                @pl.loop(0, 2 * npair)
                def _blk(j):
                    pltpu.async_copy(ht_h.at[sidx.at[j]], r0, sem0).wait()
                    pltpu.sync_copy(r0, acc.at[didx.at[j]], add=True)

        plsc.subcore_barrier()
        pltpu.sync_copy(acc.at[pl.ds(s * (C // 16), C // 16)],
                        agg_h.at[pl.ds(b * C + s * (C // 16), C // 16)])
        plsc.subcore_barrier()


def _sc_agg(ht, ls4, ld4, cnt_h):
    return pl.kernel(
        _sc_agg_body,
        out_type=jax.ShapeDtypeStruct((NPAD, D), jnp.float32),
        mesh=plsc.VectorSubcoreMesh(**_MESH),
        scratch_types=(
            pltpu.VMEM_SHARED((R, D), jnp.float32),
            pltpu.VMEM((GBLK, GB), jnp.int32),
            pltpu.VMEM((GBLK, GB), jnp.int32),
            pltpu.VMEM((GB, D), jnp.float32),
            pltpu.VMEM((GB, D), jnp.float32),
            pltpu.VMEM((16, D), jnp.float32),
            pltpu.VMEM((16,), jnp.int32),
            pltpu.SemaphoreType.DMA,
            pltpu.SemaphoreType.DMA,
        ),
        compiler_params=_SC_PARAMS,
    )(ht, ls4, ld4, cnt_h)


def _sc_pool_body(h4_h, bidx2, zeros_h, pool_h, cnt_h,
                  accp, hbuf, bbuf, cntv):
    c = lax.axis_index("c")
    s = lax.axis_index("s")
    wid = c * 16 + s

    zeros16 = jnp.zeros((16,), jnp.float32)
    ones16 = jnp.ones((16,), jnp.float32)

    pltpu.sync_copy(zeros_h.at[pl.ds(s * (PR // 16), PR // 16)],
                    accp.at[pl.ds(s * (PR // 16), PR // 16)])

    @pl.loop(0, PR // 16)
    def _zc(i):
        cntv[pl.ds(i * 16, 16)] = zeros16

    plsc.subcore_barrier()

    for k in range(13):
        blk = wid + NTILES * k
        if k == 12:
            @pl.when(blk < NROWBLK)
            def _tail():
                _sc_pool_block(h4_h, bidx2, accp, hbuf, bbuf, cntv, blk,
                               ones16)
        else:
            _sc_pool_block(h4_h, bidx2, accp, hbuf, bbuf, cntv, blk, ones16)

    plsc.subcore_barrier()
    pltpu.sync_copy(accp.at[pl.ds(s * (PR // 16), PR // 16)],
                    pool_h.at[c, pl.ds(s * (PR // 16), PR // 16)])
    pltpu.sync_copy(cntv, cnt_h.at[wid])


def _sc_pool_block(h4_h, bidx2, accp, hbuf, bbuf, cntv, blk, ones16):
    pltpu.sync_copy(h4_h.at[pl.ds(blk * 128, 128)], hbuf)
    pltpu.sync_copy(bidx2.at[blk], bbuf)
    pltpu.sync_copy(hbuf, accp.at[bbuf], add=True)
    for lane in range(8):
        iv = bbuf[pl.ds(lane * 16, 16)]
        plsc.addupdate_scatter(cntv, [iv], ones16)


def _sc_pool(h4, bidx2, zeros_h):
    return pl.kernel(
        _sc_pool_body,
        out_type=(
            jax.ShapeDtypeStruct((2, PR, D), jnp.float32),
            jax.ShapeDtypeStruct((NTILES, PR), jnp.float32),
        ),
        mesh=plsc.VectorSubcoreMesh(**_MESH),
        scratch_types=(
            pltpu.VMEM_SHARED((PR, D), jnp.float32),
            pltpu.VMEM((128, D), jnp.float32),
            pltpu.VMEM((128,), jnp.int32),
            pltpu.VMEM((PR,), jnp.float32),
        ),
        compiler_params=_SC_PARAMS,
    )(h4, bidx2, zeros_h)


def _tc0_body(x_ref, w_ref, dis_ref, out_ref):
    h = jnp.dot(x_ref[...], w_ref[...], preferred_element_type=jnp.float32)
    out_ref[...] = h * dis_ref[...]


def _tc0(x_pad, w0p, dis):
    return pl.pallas_call(
        _tc0_body,
        grid=(NPAD // 128,),
        in_specs=[
            pl.BlockSpec((128, 16), lambda i: (i, 0)),
            pl.BlockSpec((16, D), lambda i: (0, 0)),
            pl.BlockSpec((128, 1), lambda i: (i, 0)),
        ],
        out_specs=pl.BlockSpec((128, D), lambda i: (i, 0)),
        out_shape=jax.ShapeDtypeStruct((NPAD, D), jnp.float32),
    )(x_pad, w0p, dis)


def _tc_mid_body(agg_ref, ht_ref, dis_ref, bp_ref, gs_ref, be_ref, w_ref,
                 out_ref):
    dis = dis_ref[...]
    u = dis * (agg_ref[...] + ht_ref[...]) + bp_ref[...]
    v = jnp.where(u >= 0, u, 0.01 * u)
    xn = v * gs_ref[...] + be_ref[...]
    out_ref[...] = dis * jnp.dot(xn, w_ref[...],
                                 preferred_element_type=jnp.float32)


def _tc_mid(agg, ht, dis, bprev, gs, be, w):
    return pl.pallas_call(
        _tc_mid_body,
        grid=(NPAD // 128,),
        in_specs=[
            pl.BlockSpec((128, D), lambda i: (i, 0)),
            pl.BlockSpec((128, D), lambda i: (i, 0)),
            pl.BlockSpec((128, 1), lambda i: (i, 0)),
            pl.BlockSpec((1, D), lambda i: (0, 0)),
            pl.BlockSpec((1, D), lambda i: (0, 0)),
            pl.BlockSpec((1, D), lambda i: (0, 0)),
            pl.BlockSpec((D, D), lambda i: (0, 0)),
        ],
        out_specs=pl.BlockSpec((128, D), lambda i: (i, 0)),
        out_shape=jax.ShapeDtypeStruct((NPAD, D), jnp.float32),
    )(agg, ht, dis, bprev, gs, be, w)


def _tc_last_body(agg_ref, ht_ref, dis_ref, bp_ref, out_ref):
    u = dis_ref[...] * (agg_ref[...] + ht_ref[...]) + bp_ref[...]
    out_ref[...] = jnp.where(u >= 0, u, 0.01 * u)


def _tc_last(agg, ht, dis, bprev):
    return pl.pallas_call(
        _tc_last_body,
        grid=(NPAD // 128,),
        in_specs=[
            pl.BlockSpec((128, D), lambda i: (i, 0)),
            pl.BlockSpec((128, D), lambda i: (i, 0)),
            pl.BlockSpec((128, 1), lambda i: (i, 0)),
            pl.BlockSpec((1, D), lambda i: (0, 0)),
        ],
        out_specs=pl.BlockSpec((128, D), lambda i: (i, 0)),
        out_shape=jax.ShapeDtypeStruct((NPAD, D), jnp.float32),
    )(agg, ht, dis, bprev)


def _tc_head_body(pool_ref, cnt_ref, w_ref, b_ref, out_ref):
    sums = pool_ref[0, :G, :] + pool_ref[1, :G, :]
    pooled = sums / cnt_ref[...]
    out_ref[...] = jnp.dot(pooled, w_ref[...],
                           preferred_element_type=jnp.float32) + b_ref[...]


def _tc_head(pool_parts, cntc, wout, boutr):
    return pl.pallas_call(
        _tc_head_body,
        grid=(1,),
        in_specs=[
            pl.BlockSpec((2, PR, D), lambda i: (0, 0, 0)),
            pl.BlockSpec((G, 1), lambda i: (0, 0)),
            pl.BlockSpec((D, 1), lambda i: (0, 0)),
            pl.BlockSpec((1, 1), lambda i: (0, 0)),
        ],
        out_specs=pl.BlockSpec((G, 1), lambda i: (0, 0)),
        out_shape=jax.ShapeDtypeStruct((G, 1), jnp.float32),
    )(pool_parts, cntc, wout, boutr)


def kernel(x, edge_index, batch_index, W0, b0, W1, b1, W2, b2, W3, b3,
           g1, be1, g2, be2, g3, be3, Wout, bout):
    f32, i32 = jnp.float32, jnp.int32

    x_pad = jnp.pad(x, ((0, NPAD - N), (0, 16 - x.shape[1])))
    w0p = jnp.pad(W0, ((0, 16 - W0.shape[0]), (0, 0)))

    src = edge_index[0].astype(i32)
    dst = edge_index[1].astype(i32)
    dpad = N + jnp.arange(600, dtype=i32) % (NPAD - N)
    src2 = jnp.pad(src.reshape(NTILES, E // NTILES), ((0, 0), (0, 600)),
                   constant_values=0).reshape(NTILES, EROWS, 128)
    dst2 = jnp.concatenate(
        [dst.reshape(NTILES, E // NTILES),
         jnp.broadcast_to(dpad, (NTILES, 600))],
        axis=1).reshape(NTILES, EROWS, 128)

    bidx2 = jnp.concatenate(
        [batch_index.astype(i32), jnp.full((NPAD - N,), G, i32)]).reshape(
            NROWBLK, 128)

    zeros_h = jnp.zeros((PR, D), f32)

    bn_scale = 1.0 / jnp.sqrt(1.0 + 1e-5)
    b0r = b0.reshape(1, D)
    b1r = b1.reshape(1, D)
    b2r = b2.reshape(1, D)
    b3r = b3.reshape(1, D)
    gs1 = (g1 * bn_scale).reshape(1, D)
    gs2 = (g2 * bn_scale).reshape(1, D)
    gs3 = (g3 * bn_scale).reshape(1, D)
    be1r = be1.reshape(1, D)
    be2r = be2.reshape(1, D)
    be3r = be3.reshape(1, D)

    deg_parts = _sc_deg(dst2)
    ls_h, ld_h, cnt_h = _sc_buckets(src2, dst2)
    ls4 = ls_h.reshape(NTILES, NB, GBLK, GB)
    ld4 = ld_h.reshape(NTILES, NB, GBLK, GB)

    deg = jnp.sum(deg_parts, axis=0) + 1.0
    dis = jnp.where(jnp.arange(NPAD) < N, lax.rsqrt(deg), 0.0)
    dis = dis.reshape(NPAD, 1).astype(f32)

    ht = _tc0(x_pad, w0p, dis)
    agg = _sc_agg(ht, ls4, ld4, cnt_h)

    ht = _tc_mid(agg, ht, dis, b0r, gs1, be1r, W1)
    agg = _sc_agg(ht, ls4, ld4, cnt_h)

    ht = _tc_mid(agg, ht, dis, b1r, gs2, be2r, W2)
    agg = _sc_agg(ht, ls4, ld4, cnt_h)

    ht = _tc_mid(agg, ht, dis, b2r, gs3, be3r, W3)
    agg = _sc_agg(ht, ls4, ld4, cnt_h)

    h4 = _tc_last(agg, ht, dis, b3r)

    pool_parts, cnt_parts = _sc_pool(h4, bidx2, zeros_h)
    cnt = jnp.sum(cnt_parts, axis=0)[:G]
    cntc = jnp.clip(cnt, 1.0, None).reshape(G, 1)

    return _tc_head(pool_parts, cntc, Wout, bout.reshape(1, 1))


# speedup vs baseline: 1.0122x; 1.0122x over previous
import jax
import jax.numpy as jnp
from jax import lax
from jax.experimental import pallas as pl
from jax.experimental.pallas import tpu as pltpu
from jax.experimental.pallas import tpu_sc as plsc

N, E, D, G = 50000, 800000, 128, 256

NPAD = 50176
NB = 8
C = 6272
R = 6400
CAP = 5120
GB = 64
GBLK = CAP // GB
NTILES = 32
EPT = 25600
EROWS = EPT // 128
SB = 40
NROWBLK = NPAD // 128
PR = 384

_MESH = dict(core_axis_name="c", subcore_axis_name="s",
             num_cores=2, num_subcores=16)
_SC_PARAMS = pltpu.CompilerParams(needs_layout_passes=False)


def _sc_deg_body(dst2_h, deg_h, st_d, degv):
    c = lax.axis_index("c")
    s = lax.axis_index("s")
    wid = c * 16 + s
    zeros16 = jnp.zeros((16,), jnp.float32)
    ones16 = jnp.ones((16,), jnp.float32)

    @pl.loop(0, NPAD // 16)
    def _zero(i):
        degv[pl.ds(i * 16, 16)] = zeros16

    for ob in range(EROWS // SB):
        pltpu.sync_copy(dst2_h.at[wid, pl.ds(ob * SB, SB)], st_d)

        @pl.loop(0, SB)
        def _row(r):
            for lane in range(8):
                dv = st_d[r, pl.ds(lane * 16, 16)]
                plsc.addupdate_scatter(degv, [dv], ones16)

    pltpu.sync_copy(degv, deg_h.at[wid])


def _sc_deg(dst2):
    return pl.kernel(
        _sc_deg_body,
        out_type=jax.ShapeDtypeStruct((NTILES, NPAD), jnp.float32),
        mesh=plsc.VectorSubcoreMesh(**_MESH),
        scratch_types=(
            pltpu.VMEM((SB, 128), jnp.int32),
            pltpu.VMEM((NPAD,), jnp.float32),
        ),
        compiler_params=_SC_PARAMS,
    )(dst2)


def _sc_buckets_body(src2, dst2, ls_h, ld_h, cnt_h, st_s, st_d, ls, ld, cbuf):
    c = lax.axis_index("c")
    s = lax.axis_index("s")
    wid = c * 16 + s

    dummy_src = jnp.zeros((16,), jnp.int32)
    dummy_dst = jnp.full((16,), C, jnp.int32)

    offs = tuple(jnp.int32(b * CAP) for b in range(NB))

    for ob in range(EROWS // SB):
        pltpu.sync_copy(src2.at[wid, pl.ds(ob * SB, SB)], st_s)
        pltpu.sync_copy(dst2.at[wid, pl.ds(ob * SB, SB)], st_d)

        @pl.loop(0, SB, init_carry=offs)
        def _row(r, carry):
            offs_in = list(carry)
            for lane in range(8):
                sv = st_s[r, pl.ds(lane * 16, 16)]
                dv = st_d[r, pl.ds(lane * 16, 16)]
                for b in range(NB):
                    lo = b * C
                    m = (dv >= lo) & (dv < lo + C)
                    off = offs_in[b]
                    plsc.store_compressed(ls.at[pl.ds(off, 16)], sv, mask=m)
                    plsc.store_compressed(
                        ld.at[pl.ds(off, 16)], dv - lo, mask=m)
                    cnt = jnp.sum(m.astype(jnp.int32))
                    offs_in[b] = jnp.minimum(off + cnt,
                                             jnp.int32(lo + CAP - 16))
            return tuple(offs_in)

        offs = _row

    full = jnp.ones((16,), jnp.int32) >= 0
    iota16 = lax.iota(jnp.int32, 16)
    cv = jnp.zeros((16,), jnp.int32)
    for b in range(NB):
        off = offs[b]
        for k in range(256 // 16):
            o = jnp.minimum(off + k * 16, jnp.int32(b * CAP + CAP - 16))
            plsc.store_compressed(ls.at[pl.ds(o, 16)], dummy_src, mask=full)
            plsc.store_compressed(ld.at[pl.ds(o, 16)], dummy_dst, mask=full)
        cv = jnp.where(iota16 == b, jnp.full((16,), off - b * CAP), cv)
    cbuf[...] = cv

    pltpu.sync_copy(ls, ls_h.at[wid])
    pltpu.sync_copy(ld, ld_h.at[wid])
    pltpu.sync_copy(cbuf, cnt_h.at[wid])


def _sc_buckets(src2, dst2):
    return pl.kernel(
        _sc_buckets_body,
        out_type=(
            jax.ShapeDtypeStruct((NTILES, NB * CAP), jnp.int32),
            jax.ShapeDtypeStruct((NTILES, NB * CAP), jnp.int32),
            jax.ShapeDtypeStruct((NTILES, 16), jnp.int32),
        ),
        mesh=plsc.VectorSubcoreMesh(**_MESH),
        scratch_types=(
            pltpu.VMEM((SB, 128), jnp.int32),
            pltpu.VMEM((SB, 128), jnp.int32),
            pltpu.VMEM((NB * CAP,), jnp.int32),
            pltpu.VMEM((NB * CAP,), jnp.int32),
            pltpu.VMEM((16,), jnp.int32),
        ),
        compiler_params=_SC_PARAMS,
    )(src2, dst2)


def _sc_agg_body(ht_h, ls4, ld4, cnt_h, agg_h,
                 acc, sidx, didx, rbufs, zbuf, cbuf, gsems, ssems):
    c = lax.axis_index("c")
    s = lax.axis_index("s")

    zeros16 = jnp.zeros((16,), jnp.float32)
    iota16 = lax.iota(jnp.int32, 16)

    @pl.loop(0, 16 * D // 16)
    def _zb(i):
        zbuf[i // (D // 16), pl.ds((i % (D // 16)) * 16, 16)] = zeros16

    for u in range(NB // 2):
        b = (NB // 2) * c + u

        @pl.loop(0, (R // 16) // 16)
        def _z(i):
            pltpu.sync_copy(zbuf, acc.at[pl.ds(s * (R // 16) + i * 16, 16)])

        plsc.subcore_barrier()
        for u2 in range(2):
            t = 2 * s + u2
            pltpu.sync_copy(cnt_h.at[t], cbuf)
            cntv = cbuf[...]
            cnt = jnp.sum(jnp.where(iota16 == b, cntv, 0))
            nq = (cnt + (4 * GB - 1)) // (4 * GB)

            pltpu.sync_copy(ls4.at[t, b], sidx)
            pltpu.sync_copy(ld4.at[t, b], didx)

            @pl.when(nq > 0)
            def _go():
                @pl.loop(0, nq)
                def _quad(q):
                    j = 4 * q
                    gds = [
                        pltpu.async_copy(ht_h.at[sidx.at[j + k]],
                                         rbufs[k], gsems[k])
                        for k in range(4)
                    ]
                    sds = []
                    for k in range(4):
                        gds[k].wait()
                        sds.append(pltpu.async_copy(
                            rbufs[k], acc.at[didx.at[j + k]], ssems[k],
                            add=True))
                    for k in range(4):
                        sds[k].wait()

        plsc.subcore_barrier()
        pltpu.sync_copy(acc.at[pl.ds(s * (C // 16), C // 16)],
                        agg_h.at[pl.ds(b * C + s * (C // 16), C // 16)])
        plsc.subcore_barrier()


def _sc_agg(ht, ls4, ld4, cnt_h):
    return pl.kernel(
        _sc_agg_body,
        out_type=jax.ShapeDtypeStruct((NPAD, D), jnp.float32),
        mesh=plsc.VectorSubcoreMesh(**_MESH),
        scratch_types=(
            pltpu.VMEM_SHARED((R, D), jnp.float32),
            pltpu.VMEM((GBLK, GB), jnp.int32),
            pltpu.VMEM((GBLK, GB), jnp.int32),
            [pltpu.VMEM((GB, D), jnp.float32)] * 4,
            pltpu.VMEM((16, D), jnp.float32),
            pltpu.VMEM((16,), jnp.int32),
            [pltpu.SemaphoreType.DMA] * 4,
            [pltpu.SemaphoreType.DMA] * 4,
        ),
        compiler_params=_SC_PARAMS,
    )(ht, ls4, ld4, cnt_h)


def _sc_pool_body(h4_h, bidx2, zeros_h, pool_h, cnt_h,
                  accp, hbuf, bbuf, cntv):
    c = lax.axis_index("c")
    s = lax.axis_index("s")
    wid = c * 16 + s

    zeros16 = jnp.zeros((16,), jnp.float32)
    ones16 = jnp.ones((16,), jnp.float32)

    pltpu.sync_copy(zeros_h.at[pl.ds(s * (PR // 16), PR // 16)],
                    accp.at[pl.ds(s * (PR // 16), PR // 16)])

    @pl.loop(0, PR // 16)
    def _zc(i):
        cntv[pl.ds(i * 16, 16)] = zeros16

    plsc.subcore_barrier()

    for k in range(13):
        blk = wid + NTILES * k
        if k == 12:
            @pl.when(blk < NROWBLK)
            def _tail():
                _sc_pool_block(h4_h, bidx2, accp, hbuf, bbuf, cntv, blk,
                               ones16)
        else:
            _sc_pool_block(h4_h, bidx2, accp, hbuf, bbuf, cntv, blk, ones16)

    plsc.subcore_barrier()
    pltpu.sync_copy(accp.at[pl.ds(s * (PR // 16), PR // 16)],
                    pool_h.at[c, pl.ds(s * (PR // 16), PR // 16)])
    pltpu.sync_copy(cntv, cnt_h.at[wid])


def _sc_pool_block(h4_h, bidx2, accp, hbuf, bbuf, cntv, blk, ones16):
    pltpu.sync_copy(h4_h.at[pl.ds(blk * 128, 128)], hbuf)
    pltpu.sync_copy(bidx2.at[blk], bbuf)
    pltpu.sync_copy(hbuf, accp.at[bbuf], add=True)
    for lane in range(8):
        iv = bbuf[pl.ds(lane * 16, 16)]
        plsc.addupdate_scatter(cntv, [iv], ones16)


def _sc_pool(h4, bidx2, zeros_h):
    return pl.kernel(
        _sc_pool_body,
        out_type=(
            jax.ShapeDtypeStruct((2, PR, D), jnp.float32),
            jax.ShapeDtypeStruct((NTILES, PR), jnp.float32),
        ),
        mesh=plsc.VectorSubcoreMesh(**_MESH),
        scratch_types=(
            pltpu.VMEM_SHARED((PR, D), jnp.float32),
            pltpu.VMEM((128, D), jnp.float32),
            pltpu.VMEM((128,), jnp.int32),
            pltpu.VMEM((PR,), jnp.float32),
        ),
        compiler_params=_SC_PARAMS,
    )(h4, bidx2, zeros_h)


def _tc0_body(x_ref, w_ref, dis_ref, out_ref):
    h = jnp.dot(x_ref[...], w_ref[...], preferred_element_type=jnp.float32)
    out_ref[...] = h * dis_ref[...]


def _tc0(x_pad, w0p, dis):
    return pl.pallas_call(
        _tc0_body,
        grid=(NPAD // 128,),
        in_specs=[
            pl.BlockSpec((128, 16), lambda i: (i, 0)),
            pl.BlockSpec((16, D), lambda i: (0, 0)),
            pl.BlockSpec((128, 1), lambda i: (i, 0)),
        ],
        out_specs=pl.BlockSpec((128, D), lambda i: (i, 0)),
        out_shape=jax.ShapeDtypeStruct((NPAD, D), jnp.float32),
    )(x_pad, w0p, dis)


def _tc_mid_body(agg_ref, ht_ref, dis_ref, bp_ref, gs_ref, be_ref, w_ref,
                 out_ref):
    dis = dis_ref[...]
    u = dis * (agg_ref[...] + ht_ref[...]) + bp_ref[...]
    v = jnp.where(u >= 0, u, 0.01 * u)
    xn = v * gs_ref[...] + be_ref[...]
    out_ref[...] = dis * jnp.dot(xn, w_ref[...],
                                 preferred_element_type=jnp.float32)


def _tc_mid(agg, ht, dis, bprev, gs, be, w):
    return pl.pallas_call(
        _tc_mid_body,
        grid=(NPAD // 128,),
        in_specs=[
            pl.BlockSpec((128, D), lambda i: (i, 0)),
            pl.BlockSpec((128, D), lambda i: (i, 0)),
            pl.BlockSpec((128, 1), lambda i: (i, 0)),
            pl.BlockSpec((1, D), lambda i: (0, 0)),
            pl.BlockSpec((1, D), lambda i: (0, 0)),
            pl.BlockSpec((1, D), lambda i: (0, 0)),
            pl.BlockSpec((D, D), lambda i: (0, 0)),
        ],
        out_specs=pl.BlockSpec((128, D), lambda i: (i, 0)),
        out_shape=jax.ShapeDtypeStruct((NPAD, D), jnp.float32),
    )(agg, ht, dis, bprev, gs, be, w)


def _tc_last_body(agg_ref, ht_ref, dis_ref, bp_ref, out_ref):
    u = dis_ref[...] * (agg_ref[...] + ht_ref[...]) + bp_ref[...]
    out_ref[...] = jnp.where(u >= 0, u, 0.01 * u)


def _tc_last(agg, ht, dis, bprev):
    return pl.pallas_call(
        _tc_last_body,
        grid=(NPAD // 128,),
        in_specs=[
            pl.BlockSpec((128, D), lambda i: (i, 0)),
            pl.BlockSpec((128, D), lambda i: (i, 0)),
            pl.BlockSpec((128, 1), lambda i: (i, 0)),
            pl.BlockSpec((1, D), lambda i: (0, 0)),
        ],
        out_specs=pl.BlockSpec((128, D), lambda i: (i, 0)),
        out_shape=jax.ShapeDtypeStruct((NPAD, D), jnp.float32),
    )(agg, ht, dis, bprev)


def _tc_head_body(pool_ref, cnt_ref, w_ref, b_ref, out_ref):
    sums = pool_ref[0, :G, :] + pool_ref[1, :G, :]
    pooled = sums / cnt_ref[...]
    out_ref[...] = jnp.dot(pooled, w_ref[...],
                           preferred_element_type=jnp.float32) + b_ref[...]


def _tc_head(pool_parts, cntc, wout, boutr):
    return pl.pallas_call(
        _tc_head_body,
        grid=(1,),
        in_specs=[
            pl.BlockSpec((2, PR, D), lambda i: (0, 0, 0)),
            pl.BlockSpec((G, 1), lambda i: (0, 0)),
            pl.BlockSpec((D, 1), lambda i: (0, 0)),
            pl.BlockSpec((1, 1), lambda i: (0, 0)),
        ],
        out_specs=pl.BlockSpec((G, 1), lambda i: (0, 0)),
        out_shape=jax.ShapeDtypeStruct((G, 1), jnp.float32),
    )(pool_parts, cntc, wout, boutr)


def kernel(x, edge_index, batch_index, W0, b0, W1, b1, W2, b2, W3, b3,
           g1, be1, g2, be2, g3, be3, Wout, bout):
    f32, i32 = jnp.float32, jnp.int32

    x_pad = jnp.pad(x, ((0, NPAD - N), (0, 16 - x.shape[1])))
    w0p = jnp.pad(W0, ((0, 16 - W0.shape[0]), (0, 0)))

    src = edge_index[0].astype(i32)
    dst = edge_index[1].astype(i32)
    dpad = N + jnp.arange(600, dtype=i32) % (NPAD - N)
    src2 = jnp.pad(src.reshape(NTILES, E // NTILES), ((0, 0), (0, 600)),
                   constant_values=0).reshape(NTILES, EROWS, 128)
    dst2 = jnp.concatenate(
        [dst.reshape(NTILES, E // NTILES),
         jnp.broadcast_to(dpad, (NTILES, 600))],
        axis=1).reshape(NTILES, EROWS, 128)

    bidx2 = jnp.concatenate(
        [batch_index.astype(i32), jnp.full((NPAD - N,), G, i32)]).reshape(
            NROWBLK, 128)

    zeros_h = jnp.zeros((PR, D), f32)

    bn_scale = 1.0 / jnp.sqrt(1.0 + 1e-5)
    b0r = b0.reshape(1, D)
    b1r = b1.reshape(1, D)
    b2r = b2.reshape(1, D)
    b3r = b3.reshape(1, D)
    gs1 = (g1 * bn_scale).reshape(1, D)
    gs2 = (g2 * bn_scale).reshape(1, D)
    gs3 = (g3 * bn_scale).reshape(1, D)
    be1r = be1.reshape(1, D)
    be2r = be2.reshape(1, D)
    be3r = be3.reshape(1, D)

    deg_parts = _sc_deg(dst2)
    ls_h, ld_h, cnt_h = _sc_buckets(src2, dst2)
    ls4 = ls_h.reshape(NTILES, NB, GBLK, GB)
    ld4 = ld_h.reshape(NTILES, NB, GBLK, GB)

    deg = jnp.sum(deg_parts, axis=0) + 1.0
    dis = jnp.where(jnp.arange(NPAD) < N, lax.rsqrt(deg), 0.0)
    dis = dis.reshape(NPAD, 1).astype(f32)

    ht = _tc0(x_pad, w0p, dis)
    agg = _sc_agg(ht, ls4, ld4, cnt_h)

    ht = _tc_mid(agg, ht, dis, b0r, gs1, be1r, W1)
    agg = _sc_agg(ht, ls4, ld4, cnt_h)

    ht = _tc_mid(agg, ht, dis, b1r, gs2, be2r, W2)
    agg = _sc_agg(ht, ls4, ld4, cnt_h)

    ht = _tc_mid(agg, ht, dis, b2r, gs3, be3r, W3)
    agg = _sc_agg(ht, ls4, ld4, cnt_h)

    h4 = _tc_last(agg, ht, dis, b3r)

    pool_parts, cnt_parts = _sc_pool(h4, bidx2, zeros_h)
    cnt = jnp.sum(cnt_parts, axis=0)[:G]
    cntc = jnp.clip(cnt, 1.0, None).reshape(G, 1)

    return _tc_head(pool_parts, cntc, Wout, bout.reshape(1, 1))


# speedup vs baseline: 1.6572x; 1.6372x over previous
import jax
import jax.numpy as jnp
from jax import lax
from jax.experimental import pallas as pl
from jax.experimental.pallas import tpu as pltpu
from jax.experimental.pallas import tpu_sc as plsc

N, E, D, G = 50000, 800000, 128, 256

NPAD = 50176
NB = 4
C = 12544
R = 12672
CAP = 8192
GB = 64
GBLK = CAP // GB
NTILES = 32
EPT = 25600
EROWS = EPT // 128
SB = 40
NROWBLK = NPAD // 128
PR = 384

_MESH = dict(core_axis_name="c", subcore_axis_name="s",
             num_cores=2, num_subcores=16)
_SC_PARAMS = pltpu.CompilerParams(needs_layout_passes=False)


def _sc_deg_body(dst2_h, deg_h, st_d, degv):
    c = lax.axis_index("c")
    s = lax.axis_index("s")
    wid = c * 16 + s
    zeros16 = jnp.zeros((16,), jnp.float32)
    ones16 = jnp.ones((16,), jnp.float32)

    @pl.loop(0, NPAD // 16)
    def _zero(i):
        degv[pl.ds(i * 16, 16)] = zeros16

    for ob in range(EROWS // SB):
        pltpu.sync_copy(dst2_h.at[wid, pl.ds(ob * SB, SB)], st_d)

        @pl.loop(0, SB)
        def _row(r):
            for lane in range(8):
                dv = st_d[r, pl.ds(lane * 16, 16)]
                plsc.addupdate_scatter(degv, [dv], ones16)

    pltpu.sync_copy(degv, deg_h.at[wid])


def _sc_deg(dst2):
    return pl.kernel(
        _sc_deg_body,
        out_type=jax.ShapeDtypeStruct((NTILES, NPAD), jnp.float32),
        mesh=plsc.VectorSubcoreMesh(**_MESH),
        scratch_types=(
            pltpu.VMEM((SB, 128), jnp.int32),
            pltpu.VMEM((NPAD,), jnp.float32),
        ),
        compiler_params=_SC_PARAMS,
    )(dst2)


def _sc_buckets_body(src2, dst2, ls_h, ld_h, cnt_h, st_s, st_d, ls, ld, cbuf):
    c = lax.axis_index("c")
    s = lax.axis_index("s")
    wid = c * 16 + s

    dummy_src = jnp.zeros((16,), jnp.int32)
    dummy_dst = jnp.full((16,), C, jnp.int32)

    offs = tuple(jnp.int32(b * CAP) for b in range(NB))

    for ob in range(EROWS // SB):
        pltpu.sync_copy(src2.at[wid, pl.ds(ob * SB, SB)], st_s)
        pltpu.sync_copy(dst2.at[wid, pl.ds(ob * SB, SB)], st_d)

        @pl.loop(0, SB, init_carry=offs)
        def _row(r, carry):
            offs_in = list(carry)
            for lane in range(8):
                sv = st_s[r, pl.ds(lane * 16, 16)]
                dv = st_d[r, pl.ds(lane * 16, 16)]
                for b in range(NB):
                    lo = b * C
                    m = (dv >= lo) & (dv < lo + C)
                    off = offs_in[b]
                    plsc.store_compressed(ls.at[pl.ds(off, 16)], sv, mask=m)
                    plsc.store_compressed(
                        ld.at[pl.ds(off, 16)], dv - lo, mask=m)
                    cnt = jnp.sum(m.astype(jnp.int32))
                    offs_in[b] = jnp.minimum(off + cnt,
                                             jnp.int32(lo + CAP - 16))
            return tuple(offs_in)

        offs = _row

    full = jnp.ones((16,), jnp.int32) >= 0
    iota16 = lax.iota(jnp.int32, 16)
    cv = jnp.zeros((16,), jnp.int32)
    for b in range(NB):
        off = offs[b]
        for k in range(256 // 16):
            o = jnp.minimum(off + k * 16, jnp.int32(b * CAP + CAP - 16))
            plsc.store_compressed(ls.at[pl.ds(o, 16)], dummy_src, mask=full)
            plsc.store_compressed(ld.at[pl.ds(o, 16)], dummy_dst, mask=full)
        cv = jnp.where(iota16 == b, jnp.full((16,), off - b * CAP), cv)
    cbuf[...] = cv

    pltpu.sync_copy(ls, ls_h.at[wid])
    pltpu.sync_copy(ld, ld_h.at[wid])
    pltpu.sync_copy(cbuf, cnt_h.at[wid])


def _sc_buckets(src2, dst2):
    return pl.kernel(
        _sc_buckets_body,
        out_type=(
            jax.ShapeDtypeStruct((NTILES, NB * CAP), jnp.int32),
            jax.ShapeDtypeStruct((NTILES, NB * CAP), jnp.int32),
            jax.ShapeDtypeStruct((NTILES, 16), jnp.int32),
        ),
        mesh=plsc.VectorSubcoreMesh(**_MESH),
        scratch_types=(
            pltpu.VMEM((SB, 128), jnp.int32),
            pltpu.VMEM((SB, 128), jnp.int32),
            pltpu.VMEM((NB * CAP,), jnp.int32),
            pltpu.VMEM((NB * CAP,), jnp.int32),
            pltpu.VMEM((16,), jnp.int32),
        ),
        compiler_params=_SC_PARAMS,
    )(src2, dst2)


def _sc_agg_body(ht_h, ls4, ld4, cnt_h, agg_h,
                 acc, sidx, didx, rbufs, zbuf, cbuf, gsems, ssems):
    c = lax.axis_index("c")
    s = lax.axis_index("s")

    zeros16 = jnp.zeros((16,), jnp.float32)
    iota16 = lax.iota(jnp.int32, 16)

    @pl.loop(0, 24 * D // 16)
    def _zb(i):
        zbuf[i // (D // 16), pl.ds((i % (D // 16)) * 16, 16)] = zeros16

    for u in range(NB // 2):
        b = (NB // 2) * c + u

        @pl.loop(0, (R // 16) // 24)
        def _z(i):
            pltpu.sync_copy(zbuf, acc.at[pl.ds(s * (R // 16) + i * 24, 24)])

        plsc.subcore_barrier()
        for u2 in range(2):
            t = 2 * s + u2
            pltpu.sync_copy(cnt_h.at[t], cbuf)
            cntv = cbuf[...]
            cnt = jnp.sum(jnp.where(iota16 == b, cntv, 0))
            nblk = (cnt + (GB - 1)) // GB
            for hhalf in range(2):
                nh = jnp.clip(nblk - hhalf * (GBLK // 2), 0, GBLK // 2)

                @pl.when(nh > 0)
                def _go():
                    pltpu.sync_copy(
                        ls4.at[t, b, pl.ds(hhalf * (GBLK // 2), GBLK // 2)],
                        sidx)
                    pltpu.sync_copy(
                        ld4.at[t, b, pl.ds(hhalf * (GBLK // 2), GBLK // 2)],
                        didx)

                    @pl.loop(0, nh)
                    def _blk(j):
                        pltpu.async_copy(ht_h.at[sidx.at[j]], rbufs,
                                         gsems).wait()
                        pltpu.sync_copy(rbufs, acc.at[didx.at[j]], add=True)

        plsc.subcore_barrier()
        pltpu.sync_copy(acc.at[pl.ds(s * (C // 16), C // 16)],
                        agg_h.at[pl.ds(b * C + s * (C // 16), C // 16)])
        plsc.subcore_barrier()


def _sc_agg(ht, ls4, ld4, cnt_h):
    return pl.kernel(
        _sc_agg_body,
        out_type=jax.ShapeDtypeStruct((NPAD, D), jnp.float32),
        mesh=plsc.VectorSubcoreMesh(**_MESH),
        scratch_types=(
            pltpu.VMEM_SHARED((R, D), jnp.float32),
            pltpu.VMEM((GBLK // 2, GB), jnp.int32),
            pltpu.VMEM((GBLK // 2, GB), jnp.int32),
            pltpu.VMEM((GB, D), jnp.float32),
            pltpu.VMEM((24, D), jnp.float32),
            pltpu.VMEM((16,), jnp.int32),
            pltpu.SemaphoreType.DMA,
            pltpu.SemaphoreType.DMA,
        ),
        compiler_params=_SC_PARAMS,
    )(ht, ls4, ld4, cnt_h)


def _sc_pool_body(h4_h, bidx2, zeros_h, pool_h, cnt_h,
                  accp, hbuf, bbuf, cntv):
    c = lax.axis_index("c")
    s = lax.axis_index("s")
    wid = c * 16 + s

    zeros16 = jnp.zeros((16,), jnp.float32)
    ones16 = jnp.ones((16,), jnp.float32)

    pltpu.sync_copy(zeros_h.at[pl.ds(s * (PR // 16), PR // 16)],
                    accp.at[pl.ds(s * (PR // 16), PR // 16)])

    @pl.loop(0, PR // 16)
    def _zc(i):
        cntv[pl.ds(i * 16, 16)] = zeros16

    plsc.subcore_barrier()

    for k in range(13):
        blk = wid + NTILES * k
        if k == 12:
            @pl.when(blk < NROWBLK)
            def _tail():
                _sc_pool_block(h4_h, bidx2, accp, hbuf, bbuf, cntv, blk,
                               ones16)
        else:
            _sc_pool_block(h4_h, bidx2, accp, hbuf, bbuf, cntv, blk, ones16)

    plsc.subcore_barrier()
    pltpu.sync_copy(accp.at[pl.ds(s * (PR // 16), PR // 16)],
                    pool_h.at[c, pl.ds(s * (PR // 16), PR // 16)])
    pltpu.sync_copy(cntv, cnt_h.at[wid])


def _sc_pool_block(h4_h, bidx2, accp, hbuf, bbuf, cntv, blk, ones16):
    pltpu.sync_copy(h4_h.at[pl.ds(blk * 128, 128)], hbuf)
    pltpu.sync_copy(bidx2.at[blk], bbuf)
    pltpu.sync_copy(hbuf, accp.at[bbuf], add=True)
    for lane in range(8):
        iv = bbuf[pl.ds(lane * 16, 16)]
        plsc.addupdate_scatter(cntv, [iv], ones16)


def _sc_pool(h4, bidx2, zeros_h):
    return pl.kernel(
        _sc_pool_body,
        out_type=(
            jax.ShapeDtypeStruct((2, PR, D), jnp.float32),
            jax.ShapeDtypeStruct((NTILES, PR), jnp.float32),
        ),
        mesh=plsc.VectorSubcoreMesh(**_MESH),
        scratch_types=(
            pltpu.VMEM_SHARED((PR, D), jnp.float32),
            pltpu.VMEM((128, D), jnp.float32),
            pltpu.VMEM((128,), jnp.int32),
            pltpu.VMEM((PR,), jnp.float32),
        ),
        compiler_params=_SC_PARAMS,
    )(h4, bidx2, zeros_h)


def _tc0_body(x_ref, w_ref, dis_ref, out_ref):
    h = jnp.dot(x_ref[...], w_ref[...], preferred_element_type=jnp.float32)
    out_ref[...] = h * dis_ref[...]


def _tc0(x_pad, w0p, dis):
    return pl.pallas_call(
        _tc0_body,
        grid=(NPAD // 128,),
        in_specs=[
            pl.BlockSpec((128, 16), lambda i: (i, 0)),
            pl.BlockSpec((16, D), lambda i: (0, 0)),
            pl.BlockSpec((128, 1), lambda i: (i, 0)),
        ],
        out_specs=pl.BlockSpec((128, D), lambda i: (i, 0)),
        out_shape=jax.ShapeDtypeStruct((NPAD, D), jnp.float32),
    )(x_pad, w0p, dis)


def _tc_mid_body(agg_ref, ht_ref, dis_ref, bp_ref, gs_ref, be_ref, w_ref,
                 out_ref):
    dis = dis_ref[...]
    u = dis * (agg_ref[...] + ht_ref[...]) + bp_ref[...]
    v = jnp.where(u >= 0, u, 0.01 * u)
    xn = v * gs_ref[...] + be_ref[...]
    out_ref[...] = dis * jnp.dot(xn, w_ref[...],
                                 preferred_element_type=jnp.float32)


def _tc_mid(agg, ht, dis, bprev, gs, be, w):
    return pl.pallas_call(
        _tc_mid_body,
        grid=(NPAD // 128,),
        in_specs=[
            pl.BlockSpec((128, D), lambda i: (i, 0)),
            pl.BlockSpec((128, D), lambda i: (i, 0)),
            pl.BlockSpec((128, 1), lambda i: (i, 0)),
            pl.BlockSpec((1, D), lambda i: (0, 0)),
            pl.BlockSpec((1, D), lambda i: (0, 0)),
            pl.BlockSpec((1, D), lambda i: (0, 0)),
            pl.BlockSpec((D, D), lambda i: (0, 0)),
        ],
        out_specs=pl.BlockSpec((128, D), lambda i: (i, 0)),
        out_shape=jax.ShapeDtypeStruct((NPAD, D), jnp.float32),
    )(agg, ht, dis, bprev, gs, be, w)


def _tc_last_body(agg_ref, ht_ref, dis_ref, bp_ref, out_ref):
    u = dis_ref[...] * (agg_ref[...] + ht_ref[...]) + bp_ref[...]
    out_ref[...] = jnp.where(u >= 0, u, 0.01 * u)


def _tc_last(agg, ht, dis, bprev):
    return pl.pallas_call(
        _tc_last_body,
        grid=(NPAD // 128,),
        in_specs=[
            pl.BlockSpec((128, D), lambda i: (i, 0)),
            pl.BlockSpec((128, D), lambda i: (i, 0)),
            pl.BlockSpec((128, 1), lambda i: (i, 0)),
            pl.BlockSpec((1, D), lambda i: (0, 0)),
        ],
        out_specs=pl.BlockSpec((128, D), lambda i: (i, 0)),
        out_shape=jax.ShapeDtypeStruct((NPAD, D), jnp.float32),
    )(agg, ht, dis, bprev)


def _tc_head_body(pool_ref, cnt_ref, w_ref, b_ref, out_ref):
    sums = pool_ref[0, :G, :] + pool_ref[1, :G, :]
    pooled = sums / cnt_ref[...]
    out_ref[...] = jnp.dot(pooled, w_ref[...],
                           preferred_element_type=jnp.float32) + b_ref[...]


def _tc_head(pool_parts, cntc, wout, boutr):
    return pl.pallas_call(
        _tc_head_body,
        grid=(1,),
        in_specs=[
            pl.BlockSpec((2, PR, D), lambda i: (0, 0, 0)),
            pl.BlockSpec((G, 1), lambda i: (0, 0)),
            pl.BlockSpec((D, 1), lambda i: (0, 0)),
            pl.BlockSpec((1, 1), lambda i: (0, 0)),
        ],
        out_specs=pl.BlockSpec((G, 1), lambda i: (0, 0)),
        out_shape=jax.ShapeDtypeStruct((G, 1), jnp.float32),
    )(pool_parts, cntc, wout, boutr)


def kernel(x, edge_index, batch_index, W0, b0, W1, b1, W2, b2, W3, b3,
           g1, be1, g2, be2, g3, be3, Wout, bout):
    f32, i32 = jnp.float32, jnp.int32

    x_pad = jnp.pad(x, ((0, NPAD - N), (0, 16 - x.shape[1])))
    w0p = jnp.pad(W0, ((0, 16 - W0.shape[0]), (0, 0)))

    src = edge_index[0].astype(i32)
    dst = edge_index[1].astype(i32)
    dpad = N + jnp.arange(600, dtype=i32) % (NPAD - N)
    src2 = jnp.pad(src.reshape(NTILES, E // NTILES), ((0, 0), (0, 600)),
                   constant_values=0).reshape(NTILES, EROWS, 128)
    dst2 = jnp.concatenate(
        [dst.reshape(NTILES, E // NTILES),
         jnp.broadcast_to(dpad, (NTILES, 600))],
        axis=1).reshape(NTILES, EROWS, 128)

    bidx2 = jnp.concatenate(
        [batch_index.astype(i32), jnp.full((NPAD - N,), G, i32)]).reshape(
            NROWBLK, 128)

    zeros_h = jnp.zeros((PR, D), f32)

    bn_scale = 1.0 / jnp.sqrt(1.0 + 1e-5)
    b0r = b0.reshape(1, D)
    b1r = b1.reshape(1, D)
    b2r = b2.reshape(1, D)
    b3r = b3.reshape(1, D)
    gs1 = (g1 * bn_scale).reshape(1, D)
    gs2 = (g2 * bn_scale).reshape(1, D)
    gs3 = (g3 * bn_scale).reshape(1, D)
    be1r = be1.reshape(1, D)
    be2r = be2.reshape(1, D)
    be3r = be3.reshape(1, D)

    deg_parts = _sc_deg(dst2)
    ls_h, ld_h, cnt_h = _sc_buckets(src2, dst2)
    ls4 = ls_h.reshape(NTILES, NB, GBLK, GB)
    ld4 = ld_h.reshape(NTILES, NB, GBLK, GB)

    deg = jnp.sum(deg_parts, axis=0) + 1.0
    dis = jnp.where(jnp.arange(NPAD) < N, lax.rsqrt(deg), 0.0)
    dis = dis.reshape(NPAD, 1).astype(f32)

    ht = _tc0(x_pad, w0p, dis)
    agg = _sc_agg(ht, ls4, ld4, cnt_h)

    ht = _tc_mid(agg, ht, dis, b0r, gs1, be1r, W1)
    agg = _sc_agg(ht, ls4, ld4, cnt_h)

    ht = _tc_mid(agg, ht, dis, b1r, gs2, be2r, W2)
    agg = _sc_agg(ht, ls4, ld4, cnt_h)

    ht = _tc_mid(agg, ht, dis, b2r, gs3, be3r, W3)
    agg = _sc_agg(ht, ls4, ld4, cnt_h)

    h4 = _tc_last(agg, ht, dis, b3r)

    pool_parts, cnt_parts = _sc_pool(h4, bidx2, zeros_h)
    cnt = jnp.sum(cnt_parts, axis=0)[:G]
    cntc = jnp.clip(cnt, 1.0, None).reshape(G, 1)

    return _tc_head(pool_parts, cntc, Wout, bout.reshape(1, 1))
